# baseline (speedup 1.0000x reference)
import functools

import jax
import jax.numpy as jnp
from jax.experimental import pallas as pl
from jax.experimental.pallas import tpu as pltpu

_NBUF = 4
_OBUF = 2


def _sage_kernel(xf_ref, w1t_ref, w2t_ref, b_ref, adj_ref, out_ref,
                 bufs, obuf, in_sems, out_sems, *, ch):
    n = xf_ref.shape[0]
    num_chunks = n // ch

    def in_dma(k, slot):
        return pltpu.make_async_copy(
            adj_ref.at[pl.ds(k * ch, ch), :], bufs.at[slot], in_sems.at[slot])

    def out_dma(k, oslot):
        return pltpu.make_async_copy(
            obuf.at[oslot], out_ref.at[pl.ds(k * ch, ch), :], out_sems.at[oslot])

    for s in range(min(_NBUF, num_chunks)):
        in_dma(s, s).start()

    def body(k, carry):
        slot = jax.lax.rem(k, _NBUF)
        oslot = jax.lax.rem(k, _OBUF)
        in_dma(k, slot).wait()
        a = bufs[slot]
        h_n = jnp.dot(a, xf_ref[...], preferred_element_type=jnp.float32)
        acc = jnp.dot(xf_ref[pl.ds(k * ch, ch), :], w1t_ref[...],
                      preferred_element_type=jnp.float32)
        acc = acc + jnp.dot(h_n, w2t_ref[...], preferred_element_type=jnp.float32)
        res = jnp.maximum(acc + b_ref[...], 0.0)

        @pl.when(k >= _OBUF)
        def _():
            out_dma(k - _OBUF, oslot).wait()

        obuf[oslot] = res
        out_dma(k, oslot).start()

        nk = k + _NBUF

        @pl.when(nk < num_chunks)
        def _():
            in_dma(nk, slot).start()

        return carry

    jax.lax.fori_loop(0, num_chunks, body, 0)

    for t in range(max(num_chunks - _OBUF, 0), num_chunks):
        out_dma(t, t % _OBUF).wait()


def _chunk_rows(n: int) -> int:
    for cand in (200, 128, 80, 40, 16, 8):
        if n % cand == 0:
            return cand
    return n


@functools.partial(jax.jit, static_argnames=())
def kernel(x, adj, W, b):
    n, d_in = x.shape
    d_out = W.shape[0]
    w_t = W.T.astype(jnp.float32)
    w1t = w_t[:d_in]
    w2t = w_t[d_in:]
    b2 = b.reshape(1, d_out).astype(jnp.float32)

    ch = _chunk_rows(n)

    return pl.pallas_call(
        functools.partial(_sage_kernel, ch=ch),
        grid=(1,),
        in_specs=[
            pl.BlockSpec((n, d_in), lambda i: (0, 0)),
            pl.BlockSpec((d_in, d_out), lambda i: (0, 0)),
            pl.BlockSpec((d_in, d_out), lambda i: (0, 0)),
            pl.BlockSpec((1, d_out), lambda i: (0, 0)),
            pl.BlockSpec(memory_space=pl.ANY),
        ],
        out_specs=pl.BlockSpec(memory_space=pl.ANY),
        out_shape=jax.ShapeDtypeStruct((n, d_out), jnp.float32),
        scratch_shapes=[
            pltpu.VMEM((_NBUF, _chunk_rows(n), n), jnp.float32),
            pltpu.VMEM((_OBUF, _chunk_rows(n), d_out), jnp.float32),
            pltpu.SemaphoreType.DMA((_NBUF,)),
            pltpu.SemaphoreType.DMA((_OBUF,)),
        ],
    )(x, w1t, w2t, b2, adj)


# speedup vs baseline: 1.0321x; 1.0321x over previous
import functools

import jax
import jax.numpy as jnp
from jax.experimental import pallas as pl
from jax.experimental.pallas import tpu as pltpu

_NBUF = 4
_OBUF = 2


def _sage_kernel(xf_ref, w1t_ref, w2t_ref, b_ref, adj_ref, out_ref,
                 bufs, obuf, in_sems, out_sems, *, ch):
    n = xf_ref.shape[0]
    num_chunks = n // ch

    def in_dma(k, slot):
        return pltpu.make_async_copy(
            adj_ref.at[pl.ds(k * ch, ch), :], bufs.at[slot], in_sems.at[slot])

    def out_dma(k, oslot):
        return pltpu.make_async_copy(
            obuf.at[oslot], out_ref.at[pl.ds(k * ch, ch), :], out_sems.at[oslot])

    for s in range(min(_NBUF, num_chunks)):
        in_dma(s, s).start()

    def body(k, carry):
        slot = jax.lax.rem(k, _NBUF)
        oslot = jax.lax.rem(k, _OBUF)
        in_dma(k, slot).wait()
        a = bufs[slot]
        h_n = jnp.dot(a, xf_ref[...], preferred_element_type=jnp.float32)
        acc = jnp.dot(xf_ref[pl.ds(k * ch, ch), :], w1t_ref[...],
                      preferred_element_type=jnp.float32)
        acc = acc + jnp.dot(h_n, w2t_ref[...], preferred_element_type=jnp.float32)
        res = jnp.maximum(acc + b_ref[...], 0.0)

        @pl.when(k >= _OBUF)
        def _():
            out_dma(k - _OBUF, oslot).wait()

        obuf[oslot] = res
        out_dma(k, oslot).start()

        nk = k + _NBUF

        @pl.when(nk < num_chunks)
        def _():
            in_dma(nk, slot).start()

        return carry

    jax.lax.fori_loop(0, num_chunks, body, 0)

    for t in range(max(num_chunks - _OBUF, 0), num_chunks):
        out_dma(t, t % _OBUF).wait()


def _chunk_rows(n: int) -> int:
    for cand in (80, 40, 16, 8):
        if n % cand == 0:
            return cand
    return n


@functools.partial(jax.jit, static_argnames=())
def kernel(x, adj, W, b):
    n, d_in = x.shape
    d_out = W.shape[0]
    w_t = W.T.astype(jnp.float32)
    w1t = w_t[:d_in]
    w2t = w_t[d_in:]
    b2 = b.reshape(1, d_out).astype(jnp.float32)

    ch = _chunk_rows(n)

    return pl.pallas_call(
        functools.partial(_sage_kernel, ch=ch),
        grid=(1,),
        in_specs=[
            pl.BlockSpec((n, d_in), lambda i: (0, 0)),
            pl.BlockSpec((d_in, d_out), lambda i: (0, 0)),
            pl.BlockSpec((d_in, d_out), lambda i: (0, 0)),
            pl.BlockSpec((1, d_out), lambda i: (0, 0)),
            pl.BlockSpec(memory_space=pl.ANY),
        ],
        out_specs=pl.BlockSpec(memory_space=pl.ANY),
        out_shape=jax.ShapeDtypeStruct((n, d_out), jnp.float32),
        scratch_shapes=[
            pltpu.VMEM((_NBUF, _chunk_rows(n), n), jnp.float32),
            pltpu.VMEM((_OBUF, _chunk_rows(n), d_out), jnp.float32),
            pltpu.SemaphoreType.DMA((_NBUF,)),
            pltpu.SemaphoreType.DMA((_OBUF,)),
        ],
    )(x, w1t, w2t, b2, adj)
